# baseline (speedup 1.0000x reference)
import functools

import jax
import jax.numpy as jnp
from jax import lax
from jax.experimental import pallas as pl
from jax.experimental.pallas import tpu as pltpu
from jax.experimental.pallas import tpu_sc as plsc

D = 1024
E = 8
FF = 4096
T = 2048
TB = 512
BLK = 256
NB = 24
NPAD = NB * BLK
NF2 = 2
FT2 = FF // NF2

NC = 2
NS = 16
NW = NC * NS
RPT = NPAD // NW
GCH = 48
TPT = T // NW
CCH = 8


def _gelu_exact(v):
    return v * 0.5 * (1.0 + lax.erf(v * 0.7071067811865476))


def _prep_kernel(x_ref, gw_ref, gb_ref, gamma_ref, beta_ref, xn_ref, lg_ref):
    xb = x_ref[...]
    mu = jnp.mean(xb, axis=-1, keepdims=True)
    var = jnp.mean((xb - mu) ** 2, axis=-1, keepdims=True)
    xn = (xb - mu) / jnp.sqrt(var + 1e-5) * gamma_ref[...] + beta_ref[...]
    xn_ref[...] = xn.astype(jnp.bfloat16)
    lg_ref[...] = jnp.dot(xn, gw_ref[...],
                          preferred_element_type=jnp.float32) + gb_ref[...]


def _route_kernel(lg_ref, mi_ref, be_ref, wb0_ref, wb1_ref, run_ref, re_ref):
    logits = lg_ref[...]
    m = jnp.max(logits, axis=-1, keepdims=True)
    ex = jnp.exp(logits - m)
    probs = ex / jnp.sum(ex, axis=-1, keepdims=True)
    lane = lax.broadcasted_iota(jnp.int32, (T, E), 1)
    m0 = jnp.max(probs, axis=-1, keepdims=True)
    e0 = jnp.min(jnp.where(probs == m0, lane, E), axis=-1, keepdims=True)
    probs1 = jnp.where(lane == e0, -1.0, probs)
    m1 = jnp.max(probs1, axis=-1, keepdims=True)
    e1 = jnp.min(jnp.where(probs1 == m1, lane, E), axis=-1, keepdims=True)
    denom = m0 + m1 + 1e-8
    w0 = m0 / denom
    w1 = m1 / denom

    oh0 = (lane == e0).astype(jnp.float32)
    oh1 = (lane == e1).astype(jnp.float32)
    a01 = oh0 + oh1
    def shift(v, k):
        return jnp.concatenate(
            [jnp.zeros((k, E), jnp.float32), v[:T - k, :]], axis=0)
    cnt = shift(a01, 1)
    k = 1
    while k < T:
        cnt = cnt + shift(cnt, k)
        k *= 2

    counts = jnp.sum(a01, axis=0, keepdims=True)
    rounded = jnp.ceil(counts / BLK) * BLK
    s = rounded
    for sh in (1, 2, 4):
        s = s + jnp.concatenate(
            [jnp.zeros((1, sh), jnp.float32), s[:, :E - sh]], axis=1)
    pstart = s - rounded

    pos = pstart + cnt
    pos0 = jnp.sum(jnp.where(lane == e0, pos, 0.0), axis=-1, keepdims=True)
    pos1 = jnp.sum(jnp.where(lane == e1, pos, 0.0), axis=-1, keepdims=True)
    mi_ref[...] = jnp.where(lane == 0, pos0.astype(jnp.int32),
                            jnp.where(lane == 1, pos1.astype(jnp.int32), 0))
    ones = jnp.ones((1, 128), jnp.float32)
    wb0_ref[...] = w0 * ones
    wb1_ref[...] = w1 * ones

    pend = pstart + rounded
    bi = (lax.broadcasted_iota(jnp.int32, (NB, 1), 0) * BLK
          ).astype(jnp.float32)
    be = jnp.sum((pend <= bi).astype(jnp.int32), axis=-1, keepdims=True)
    be_ref[...] = be

    eb = jnp.minimum(be, E - 1)
    prev = jnp.concatenate([eb[:1, :], eb[:NB - 1, :]], axis=0)
    chg = (eb != prev).astype(jnp.int32)
    run = chg
    k = 1
    while k < NB:
        run = run + jnp.concatenate(
            [jnp.zeros((k, 1), jnp.int32), run[:NB - k, :]], axis=0)
        k *= 2
    run_ref[...] = run
    first = jnp.concatenate(
        [jnp.ones((1, 1), jnp.int32), chg[1:, :]], axis=0)
    lanej = lax.broadcasted_iota(jnp.int32, (NB, NB), 1)
    msk = (lanej == run).astype(jnp.int32) * first
    re_ref[...] = jnp.sum(msk * eb, axis=0, keepdims=True)


def _dispatch_kernel(xn_hbm, p0_hbm, p1_hbm, wb0_hbm, wb1_hbm,
                     xs_hbm, ws_hbm,
                     p0_v, p1_v, rows_v, wr0_v, wr1_v, sems):
    cid = lax.axis_index("c")
    sid = lax.axis_index("s")
    wid = sid * NC + cid
    base = wid * TPT

    pltpu.sync_copy(p0_hbm.at[pl.ds(base, TPT)], p0_v)
    pltpu.sync_copy(p1_hbm.at[pl.ds(base, TPT)], p1_v)
    pltpu.sync_copy(xn_hbm.at[pl.ds(base, TPT)], rows_v)
    pltpu.sync_copy(wb0_hbm.at[pl.ds(base, TPT)], wr0_v)
    pltpu.sync_copy(wb1_hbm.at[pl.ds(base, TPT)], wr1_v)
    cps = [
        pltpu.async_copy(rows_v, xs_hbm.at[p0_v], sems[0]),
        pltpu.async_copy(rows_v, xs_hbm.at[p1_v], sems[1]),
        pltpu.async_copy(wr0_v, ws_hbm.at[p0_v], sems[2]),
        pltpu.async_copy(wr1_v, ws_hbm.at[p1_v], sems[3]),
    ]
    for c in cps:
        c.wait()


def _ffn_body(f, be_ref, run_ref, re_ref, xs_ref, w1_any, b1_ref, w2_any,
              extra_ref, ws_ref, out_ref, w1buf, w2buf, w1sems, w2sems,
              is_pass0):
    b = pl.program_id(0)
    r = run_ref[b]
    slot = lax.rem(r, 2)

    def start_load(rr, slot_):
        e = re_ref[jnp.minimum(rr, NB - 1)]
        pltpu.make_async_copy(w1_any.at[e, :, pl.ds(f * FT2, FT2)],
                              w1buf.at[slot_], w1sems.at[slot_]).start()
        pltpu.make_async_copy(w2_any.at[e, pl.ds(f * FT2, FT2), :],
                              w2buf.at[slot_], w2sems.at[slot_]).start()

    def wait_load(slot_):
        pltpu.make_async_copy(w1_any.at[0, :, pl.ds(f * FT2, FT2)],
                              w1buf.at[slot_], w1sems.at[slot_]).wait()
        pltpu.make_async_copy(w2_any.at[0, pl.ds(f * FT2, FT2), :],
                              w2buf.at[slot_], w2sems.at[slot_]).wait()

    @pl.when(b == 0)
    def _():
        start_load(0, 0)

    first = jnp.logical_or(b == 0, run_ref[jnp.maximum(b - 1, 0)] != r)

    @pl.when(first)
    def _():
        wait_load(slot)
        start_load(r + 1, 1 - slot)

    def compute(w1r, w2r):
        xb = xs_ref[...]
        h = jnp.dot(xb, w1r[...].astype(jnp.bfloat16),
                    preferred_element_type=jnp.float32) + b1_ref[0]
        h = _gelu_exact(h).astype(jnp.bfloat16)
        y = jnp.dot(h, w2r[...].astype(jnp.bfloat16),
                    preferred_element_type=jnp.float32)
        if is_pass0:
            out_ref[...] = ws_ref[0][:, 0:1] * (y + extra_ref[0])
        else:
            out_ref[...] = extra_ref[...] + ws_ref[0][:, 0:1] * y

    live = be_ref[b] < E

    @pl.when(jnp.logical_and(live, slot == 0))
    def _():
        compute(w1buf.at[0], w2buf.at[0])

    @pl.when(jnp.logical_and(live, slot == 1))
    def _():
        compute(w1buf.at[1], w2buf.at[1])

    @pl.when(b == NB - 1)
    def _():
        wait_load(1 - slot)


def _ffn_kernel0(*args):
    _ffn_body(0, *args, is_pass0=True)


def _ffn_kernel1(*args):
    _ffn_body(1, *args, is_pass0=False)


def _combine_kernel(x_hbm, y_hbm, p0_hbm, p1_hbm, out_hbm,
                    p0_v, p1_v,
                    y0a, y1a, xa, oa, y0b, y1b, xb, ob, sems):
    cid = lax.axis_index("c")
    sid = lax.axis_index("s")
    wid = sid * NC + cid
    base = wid * TPT

    pltpu.sync_copy(p0_hbm.at[pl.ds(base, TPT)], p0_v)
    pltpu.sync_copy(p1_hbm.at[pl.ds(base, TPT)], p1_v)

    bufs = ((y0a, y1a, xa, oa), (y0b, y1b, xb, ob))
    ncp = TPT // CCH
    incp = [None, None]
    outcp = [None, None]

    def start_in(ch):
        b = ch & 1
        o = ch * CCH
        y0_v, y1_v, x_v, _ = bufs[b]
        incp[b] = [
            pltpu.async_copy(y_hbm.at[p0_v.at[pl.ds(o, CCH)]], y0_v,
                             sems[3 * b]),
            pltpu.async_copy(y_hbm.at[p1_v.at[pl.ds(o, CCH)]], y1_v,
                             sems[3 * b + 1]),
            pltpu.async_copy(x_hbm.at[pl.ds(base + o, CCH)], x_v,
                             sems[3 * b + 2]),
        ]

    start_in(0)
    for ch in range(ncp):
        b = ch & 1
        y0_v, y1_v, x_v, o_v = bufs[b]
        for cpd in incp[b]:
            cpd.wait()
        if ch + 1 < ncp:
            start_in(ch + 1)
        if outcp[b] is not None:
            outcp[b].wait()

        def sum_body(v, _):
            sl = pl.ds(v * 16, 16)
            for r in range(CCH):
                o_v[r, sl] = x_v[r, sl] + (y0_v[r, sl] + y1_v[r, sl])
            return 0
        lax.fori_loop(0, D // 16, sum_body, 0)
        outcp[b] = pltpu.async_copy(
            o_v, out_hbm.at[pl.ds(base + ch * CCH, CCH)], sems[6 + b])
    for b in range(2):
        if outcp[b] is not None:
            outcp[b].wait()


def kernel(x, gate_W, gate_b, W1, b1, W2, b2, gamma, beta):
    b, s, d = x.shape
    flat = x.reshape(-1, d)

    xn, logits = pl.pallas_call(
        _prep_kernel,
        grid=(T // TB,),
        in_specs=[
            pl.BlockSpec((TB, D), lambda i: (i, 0)),
            pl.BlockSpec((D, E), lambda i: (0, 0)),
            pl.BlockSpec((E,), lambda i: (0,)),
            pl.BlockSpec((D,), lambda i: (0,)),
            pl.BlockSpec((D,), lambda i: (0,)),
        ],
        out_specs=[
            pl.BlockSpec((TB, D), lambda i: (i, 0)),
            pl.BlockSpec((TB, E), lambda i: (i, 0)),
        ],
        out_shape=[
            jax.ShapeDtypeStruct((T, D), jnp.bfloat16),
            jax.ShapeDtypeStruct((T, E), jnp.float32),
        ],
    )(flat, gate_W, gate_b, gamma, beta)

    mi, be, wb0, wb1, run_o, re_o = pl.pallas_call(
        _route_kernel,
        out_shape=[
            jax.ShapeDtypeStruct((T, E), jnp.int32),
            jax.ShapeDtypeStruct((NB, 1), jnp.int32),
            jax.ShapeDtypeStruct((T, 128), jnp.float32),
            jax.ShapeDtypeStruct((T, 128), jnp.float32),
            jax.ShapeDtypeStruct((NB, 1), jnp.int32),
            jax.ShapeDtypeStruct((1, NB), jnp.int32),
        ],
    )(logits)

    pos0 = mi[:, 0]
    pos1 = mi[:, 1]
    blk_e = be.reshape(NB)
    run_a = run_o.reshape(NB)
    re_a = re_o.reshape(NB)

    mesh = plsc.VectorSubcoreMesh(core_axis_name="c", subcore_axis_name="s")
    xs3, ws_wide = pl.kernel(
        _dispatch_kernel,
        out_type=[
            jax.ShapeDtypeStruct((NPAD, 8, 128), jnp.bfloat16),
            jax.ShapeDtypeStruct((NPAD, 128), jnp.float32),
        ],
        mesh=mesh,
        compiler_params=pltpu.CompilerParams(use_tc_tiling_on_sc=False),
        scratch_types=[
            pltpu.VMEM((TPT,), jnp.int32),
            pltpu.VMEM((TPT,), jnp.int32),
            pltpu.VMEM((TPT, 8, 128), jnp.bfloat16),
            pltpu.VMEM((TPT, 128), jnp.float32),
            pltpu.VMEM((TPT, 128), jnp.float32),
            [pltpu.SemaphoreType.DMA] * 4,
        ],
    )(xn.reshape(T, 8, 128), pos0, pos1, wb0, wb1)
    xs = xs3.reshape(NPAD, D)

    def emin(be_r, b):
        return jnp.minimum(be_r[b], E - 1)

    b1r = b1.reshape(E * NF2, 1, FT2)
    wsr = ws_wide.reshape(NB, BLK, 128)
    ffn_scratch = [
        pltpu.VMEM((2, D, FT2), jnp.float32),
        pltpu.VMEM((2, FT2, D), jnp.float32),
        pltpu.SemaphoreType.DMA((2,)),
        pltpu.SemaphoreType.DMA((2,)),
    ]
    grid_spec0 = pltpu.PrefetchScalarGridSpec(
        num_scalar_prefetch=3,
        grid=(NB,),
        in_specs=[
            pl.BlockSpec((BLK, D), lambda b, be_r, ru, re: (b, 0)),
            pl.BlockSpec(memory_space=pl.ANY),
            pl.BlockSpec((1, 1, FT2),
                         lambda b, be_r, ru, re: (emin(be_r, b) * NF2, 0, 0)),
            pl.BlockSpec(memory_space=pl.ANY),
            pl.BlockSpec((1, 1, D),
                         lambda b, be_r, ru, re: (emin(be_r, b), 0, 0)),
            pl.BlockSpec((1, BLK, 128), lambda b, be_r, ru, re: (b, 0, 0)),
        ],
        out_specs=pl.BlockSpec((BLK, D), lambda b, be_r, ru, re: (b, 0)),
        scratch_shapes=ffn_scratch,
    )
    yp0 = pl.pallas_call(
        _ffn_kernel0,
        grid_spec=grid_spec0,
        out_shape=jax.ShapeDtypeStruct((NPAD, D), jnp.float32),
        compiler_params=pltpu.CompilerParams(
            dimension_semantics=("arbitrary",),
        ),
    )(blk_e, run_a, re_a, xs, W1, b1r, W2, b2.reshape(E, 1, D), wsr)

    grid_spec1 = pltpu.PrefetchScalarGridSpec(
        num_scalar_prefetch=3,
        grid=(NB,),
        in_specs=[
            pl.BlockSpec((BLK, D), lambda b, be_r, ru, re: (b, 0)),
            pl.BlockSpec(memory_space=pl.ANY),
            pl.BlockSpec((1, 1, FT2),
                         lambda b, be_r, ru, re: (emin(be_r, b) * NF2 + 1,
                                                  0, 0)),
            pl.BlockSpec(memory_space=pl.ANY),
            pl.BlockSpec((BLK, D), lambda b, be_r, ru, re: (b, 0)),
            pl.BlockSpec((1, BLK, 128), lambda b, be_r, ru, re: (b, 0, 0)),
        ],
        out_specs=pl.BlockSpec((BLK, D), lambda b, be_r, ru, re: (b, 0)),
        scratch_shapes=ffn_scratch,
    )
    yp = pl.pallas_call(
        _ffn_kernel1,
        grid_spec=grid_spec1,
        out_shape=jax.ShapeDtypeStruct((NPAD, D), jnp.float32),
        compiler_params=pltpu.CompilerParams(
            dimension_semantics=("arbitrary",),
        ),
    )(blk_e, run_a, re_a, xs, W1, b1r, W2, yp0, wsr)

    out = pl.kernel(
        _combine_kernel,
        out_type=jax.ShapeDtypeStruct((T, D), jnp.float32),
        mesh=mesh,
        scratch_types=[
            pltpu.VMEM((TPT,), jnp.int32),
            pltpu.VMEM((TPT,), jnp.int32),
            pltpu.VMEM((CCH, D), jnp.float32),
            pltpu.VMEM((CCH, D), jnp.float32),
            pltpu.VMEM((CCH, D), jnp.float32),
            pltpu.VMEM((CCH, D), jnp.float32),
            pltpu.VMEM((CCH, D), jnp.float32),
            pltpu.VMEM((CCH, D), jnp.float32),
            pltpu.VMEM((CCH, D), jnp.float32),
            pltpu.VMEM((CCH, D), jnp.float32),
            [pltpu.SemaphoreType.DMA] * 8,
        ],
    )(flat, yp, pos0, pos1)

    return out.reshape(b, s, d)


# speedup vs baseline: 1.3007x; 1.3007x over previous
import functools

import jax
import jax.numpy as jnp
from jax import lax
from jax.experimental import pallas as pl
from jax.experimental.pallas import tpu as pltpu
from jax.experimental.pallas import tpu_sc as plsc

D = 1024
E = 8
FF = 4096
T = 2048
TB = 512
BLK = 256
NB = 24
NPAD = NB * BLK
NF2 = 2
FT2 = FF // NF2

NC = 2
NS = 16
NW = NC * NS
RPT = NPAD // NW
GCH = 48
TPT = T // NW
CCH = 8


def _gelu_exact(v):
    return v * 0.5 * (1.0 + lax.erf(v * 0.7071067811865476))


def _prep_kernel(x_ref, gw_ref, gb_ref, gamma_ref, beta_ref, xn_ref, lg_ref):
    xb = x_ref[...]
    mu = jnp.mean(xb, axis=-1, keepdims=True)
    var = jnp.mean((xb - mu) ** 2, axis=-1, keepdims=True)
    xn = (xb - mu) / jnp.sqrt(var + 1e-5) * gamma_ref[...] + beta_ref[...]
    xn_ref[...] = xn
    lg_ref[...] = jnp.dot(xn, gw_ref[...],
                          preferred_element_type=jnp.float32) + gb_ref[...]


def _route_kernel(lg_ref, mi_ref, be_ref, wb0_ref, wb1_ref, run_ref, re_ref):
    logits = lg_ref[...]
    m = jnp.max(logits, axis=-1, keepdims=True)
    ex = jnp.exp(logits - m)
    probs = ex / jnp.sum(ex, axis=-1, keepdims=True)
    lane = lax.broadcasted_iota(jnp.int32, (T, E), 1)
    m0 = jnp.max(probs, axis=-1, keepdims=True)
    e0 = jnp.min(jnp.where(probs == m0, lane, E), axis=-1, keepdims=True)
    probs1 = jnp.where(lane == e0, -1.0, probs)
    m1 = jnp.max(probs1, axis=-1, keepdims=True)
    e1 = jnp.min(jnp.where(probs1 == m1, lane, E), axis=-1, keepdims=True)
    denom = m0 + m1 + 1e-8
    w0 = m0 / denom
    w1 = m1 / denom

    oh0 = (lane == e0).astype(jnp.float32)
    oh1 = (lane == e1).astype(jnp.float32)
    a01 = oh0 + oh1
    def shift(v, k):
        return jnp.concatenate(
            [jnp.zeros((k, E), jnp.float32), v[:T - k, :]], axis=0)
    cnt = shift(a01, 1)
    k = 1
    while k < T:
        cnt = cnt + shift(cnt, k)
        k *= 2

    counts = jnp.sum(a01, axis=0, keepdims=True)
    rounded = jnp.ceil(counts / BLK) * BLK
    s = rounded
    for sh in (1, 2, 4):
        s = s + jnp.concatenate(
            [jnp.zeros((1, sh), jnp.float32), s[:, :E - sh]], axis=1)
    pstart = s - rounded

    pos = pstart + cnt
    pos0 = jnp.sum(jnp.where(lane == e0, pos, 0.0), axis=-1, keepdims=True)
    pos1 = jnp.sum(jnp.where(lane == e1, pos, 0.0), axis=-1, keepdims=True)
    mi_ref[...] = jnp.where(lane == 0, pos0.astype(jnp.int32),
                            jnp.where(lane == 1, pos1.astype(jnp.int32), 0))
    ones = jnp.ones((1, 128), jnp.float32)
    wb0_ref[...] = w0 * ones
    wb1_ref[...] = w1 * ones

    pend = pstart + rounded
    bi = (lax.broadcasted_iota(jnp.int32, (NB, 1), 0) * BLK
          ).astype(jnp.float32)
    be = jnp.sum((pend <= bi).astype(jnp.int32), axis=-1, keepdims=True)
    be_ref[...] = be

    eb = jnp.minimum(be, E - 1)
    prev = jnp.concatenate([eb[:1, :], eb[:NB - 1, :]], axis=0)
    chg = (eb != prev).astype(jnp.int32)
    run = chg
    k = 1
    while k < NB:
        run = run + jnp.concatenate(
            [jnp.zeros((k, 1), jnp.int32), run[:NB - k, :]], axis=0)
        k *= 2
    run_ref[...] = run
    first = jnp.concatenate(
        [jnp.ones((1, 1), jnp.int32), chg[1:, :]], axis=0)
    lanej = lax.broadcasted_iota(jnp.int32, (NB, NB), 1)
    msk = (lanej == run).astype(jnp.int32) * first
    re_ref[...] = jnp.sum(msk * eb, axis=0, keepdims=True)


def _dispatch_kernel(xn_hbm, p0_hbm, p1_hbm, wb0_hbm, wb1_hbm,
                     xs_hbm, ws_hbm,
                     p0_v, p1_v, rows_v, wr0_v, wr1_v, sems):
    cid = lax.axis_index("c")
    sid = lax.axis_index("s")
    wid = sid * NC + cid
    base = wid * TPT

    pltpu.sync_copy(p0_hbm.at[pl.ds(base, TPT)], p0_v)
    pltpu.sync_copy(p1_hbm.at[pl.ds(base, TPT)], p1_v)
    pltpu.sync_copy(xn_hbm.at[pl.ds(base, TPT)], rows_v)
    pltpu.sync_copy(wb0_hbm.at[pl.ds(base, TPT)], wr0_v)
    pltpu.sync_copy(wb1_hbm.at[pl.ds(base, TPT)], wr1_v)
    cps = [
        pltpu.async_copy(rows_v, xs_hbm.at[p0_v], sems[0]),
        pltpu.async_copy(rows_v, xs_hbm.at[p1_v], sems[1]),
        pltpu.async_copy(wr0_v, ws_hbm.at[p0_v], sems[2]),
        pltpu.async_copy(wr1_v, ws_hbm.at[p1_v], sems[3]),
    ]
    for c in cps:
        c.wait()


def _ffn_body(f, be_ref, run_ref, re_ref, xs_ref, w1_any, b1_ref, w2_any,
              extra_ref, ws_ref, out_ref, w1buf, w2buf, w1sems, w2sems,
              is_pass0):
    b = pl.program_id(0)
    r = run_ref[b]
    slot = lax.rem(r, 2)

    def start_load(rr, slot_):
        e = re_ref[jnp.minimum(rr, NB - 1)]
        pltpu.make_async_copy(w1_any.at[e, :, pl.ds(f * FT2, FT2)],
                              w1buf.at[slot_], w1sems.at[slot_]).start()
        pltpu.make_async_copy(w2_any.at[e, pl.ds(f * FT2, FT2), :],
                              w2buf.at[slot_], w2sems.at[slot_]).start()

    def wait_load(slot_):
        pltpu.make_async_copy(w1_any.at[0, :, pl.ds(f * FT2, FT2)],
                              w1buf.at[slot_], w1sems.at[slot_]).wait()
        pltpu.make_async_copy(w2_any.at[0, pl.ds(f * FT2, FT2), :],
                              w2buf.at[slot_], w2sems.at[slot_]).wait()

    @pl.when(b == 0)
    def _():
        start_load(0, 0)

    first = jnp.logical_or(b == 0, run_ref[jnp.maximum(b - 1, 0)] != r)

    @pl.when(first)
    def _():
        wait_load(slot)
        start_load(r + 1, 1 - slot)

    def compute(w1r, w2r):
        xb = xs_ref[...].astype(jnp.bfloat16)
        h = jnp.dot(xb, w1r[...].astype(jnp.bfloat16),
                    preferred_element_type=jnp.float32) + b1_ref[0]
        h = _gelu_exact(h).astype(jnp.bfloat16)
        y = jnp.dot(h, w2r[...].astype(jnp.bfloat16),
                    preferred_element_type=jnp.float32)
        if is_pass0:
            out_ref[...] = ws_ref[0][:, 0:1] * (y + extra_ref[0])
        else:
            out_ref[...] = extra_ref[...] + ws_ref[0][:, 0:1] * y

    live = be_ref[b] < E

    @pl.when(jnp.logical_and(live, slot == 0))
    def _():
        compute(w1buf.at[0], w2buf.at[0])

    @pl.when(jnp.logical_and(live, slot == 1))
    def _():
        compute(w1buf.at[1], w2buf.at[1])

    @pl.when(b == NB - 1)
    def _():
        wait_load(1 - slot)


def _ffn_kernel0(*args):
    _ffn_body(0, *args, is_pass0=True)


def _ffn_kernel1(*args):
    _ffn_body(1, *args, is_pass0=False)


def _combine_kernel(x_hbm, y_hbm, p0_hbm, p1_hbm, out_hbm,
                    p0_v, p1_v,
                    y0a, y1a, xa, oa, y0b, y1b, xb, ob, sems):
    cid = lax.axis_index("c")
    sid = lax.axis_index("s")
    wid = sid * NC + cid
    base = wid * TPT

    pltpu.sync_copy(p0_hbm.at[pl.ds(base, TPT)], p0_v)
    pltpu.sync_copy(p1_hbm.at[pl.ds(base, TPT)], p1_v)

    bufs = ((y0a, y1a, xa, oa), (y0b, y1b, xb, ob))
    ncp = TPT // CCH
    incp = [None, None]
    outcp = [None, None]

    def start_in(ch):
        b = ch & 1
        o = ch * CCH
        y0_v, y1_v, x_v, _ = bufs[b]
        incp[b] = [
            pltpu.async_copy(y_hbm.at[p0_v.at[pl.ds(o, CCH)]], y0_v,
                             sems[3 * b]),
            pltpu.async_copy(y_hbm.at[p1_v.at[pl.ds(o, CCH)]], y1_v,
                             sems[3 * b + 1]),
            pltpu.async_copy(x_hbm.at[pl.ds(base + o, CCH)], x_v,
                             sems[3 * b + 2]),
        ]

    start_in(0)
    for ch in range(ncp):
        b = ch & 1
        y0_v, y1_v, x_v, o_v = bufs[b]
        for cpd in incp[b]:
            cpd.wait()
        if ch + 1 < ncp:
            start_in(ch + 1)
        if outcp[b] is not None:
            outcp[b].wait()

        def sum_body(v, _):
            sl = pl.ds(v * 16, 16)
            for r in range(CCH):
                o_v[r, sl] = x_v[r, sl] + (y0_v[r, sl] + y1_v[r, sl])
            return 0
        lax.fori_loop(0, D // 16, sum_body, 0)
        outcp[b] = pltpu.async_copy(
            o_v, out_hbm.at[pl.ds(base + ch * CCH, CCH)], sems[6 + b])
    for b in range(2):
        if outcp[b] is not None:
            outcp[b].wait()


def kernel(x, gate_W, gate_b, W1, b1, W2, b2, gamma, beta):
    b, s, d = x.shape
    flat = x.reshape(-1, d)

    xn, logits = pl.pallas_call(
        _prep_kernel,
        grid=(T // TB,),
        in_specs=[
            pl.BlockSpec((TB, D), lambda i: (i, 0)),
            pl.BlockSpec((D, E), lambda i: (0, 0)),
            pl.BlockSpec((E,), lambda i: (0,)),
            pl.BlockSpec((D,), lambda i: (0,)),
            pl.BlockSpec((D,), lambda i: (0,)),
        ],
        out_specs=[
            pl.BlockSpec((TB, D), lambda i: (i, 0)),
            pl.BlockSpec((TB, E), lambda i: (i, 0)),
        ],
        out_shape=[
            jax.ShapeDtypeStruct((T, D), jnp.float32),
            jax.ShapeDtypeStruct((T, E), jnp.float32),
        ],
    )(flat, gate_W, gate_b, gamma, beta)

    mi, be, wb0, wb1, run_o, re_o = pl.pallas_call(
        _route_kernel,
        out_shape=[
            jax.ShapeDtypeStruct((T, E), jnp.int32),
            jax.ShapeDtypeStruct((NB, 1), jnp.int32),
            jax.ShapeDtypeStruct((T, 128), jnp.float32),
            jax.ShapeDtypeStruct((T, 128), jnp.float32),
            jax.ShapeDtypeStruct((NB, 1), jnp.int32),
            jax.ShapeDtypeStruct((1, NB), jnp.int32),
        ],
    )(logits)

    pos0 = mi[:, 0]
    pos1 = mi[:, 1]
    blk_e = be.reshape(NB)
    run_a = run_o.reshape(NB)
    re_a = re_o.reshape(NB)

    mesh = plsc.VectorSubcoreMesh(core_axis_name="c", subcore_axis_name="s")
    xs, ws_wide = pl.kernel(
        _dispatch_kernel,
        out_type=[
            jax.ShapeDtypeStruct((NPAD, D), jnp.float32),
            jax.ShapeDtypeStruct((NPAD, 128), jnp.float32),
        ],
        mesh=mesh,
        scratch_types=[
            pltpu.VMEM((TPT,), jnp.int32),
            pltpu.VMEM((TPT,), jnp.int32),
            pltpu.VMEM((TPT, D), jnp.float32),
            pltpu.VMEM((TPT, 128), jnp.float32),
            pltpu.VMEM((TPT, 128), jnp.float32),
            [pltpu.SemaphoreType.DMA] * 4,
        ],
    )(xn, pos0, pos1, wb0, wb1)

    def emin(be_r, b):
        return jnp.minimum(be_r[b], E - 1)

    b1r = b1.reshape(E * NF2, 1, FT2)
    wsr = ws_wide.reshape(NB, BLK, 128)
    ffn_scratch = [
        pltpu.VMEM((2, D, FT2), jnp.float32),
        pltpu.VMEM((2, FT2, D), jnp.float32),
        pltpu.SemaphoreType.DMA((2,)),
        pltpu.SemaphoreType.DMA((2,)),
    ]
    grid_spec0 = pltpu.PrefetchScalarGridSpec(
        num_scalar_prefetch=3,
        grid=(NB,),
        in_specs=[
            pl.BlockSpec((BLK, D), lambda b, be_r, ru, re: (b, 0)),
            pl.BlockSpec(memory_space=pl.ANY),
            pl.BlockSpec((1, 1, FT2),
                         lambda b, be_r, ru, re: (emin(be_r, b) * NF2, 0, 0)),
            pl.BlockSpec(memory_space=pl.ANY),
            pl.BlockSpec((1, 1, D),
                         lambda b, be_r, ru, re: (emin(be_r, b), 0, 0)),
            pl.BlockSpec((1, BLK, 128), lambda b, be_r, ru, re: (b, 0, 0)),
        ],
        out_specs=pl.BlockSpec((BLK, D), lambda b, be_r, ru, re: (b, 0)),
        scratch_shapes=ffn_scratch,
    )
    yp0 = pl.pallas_call(
        _ffn_kernel0,
        grid_spec=grid_spec0,
        out_shape=jax.ShapeDtypeStruct((NPAD, D), jnp.float32),
        compiler_params=pltpu.CompilerParams(
            dimension_semantics=("arbitrary",),
        ),
    )(blk_e, run_a, re_a, xs, W1, b1r, W2, b2.reshape(E, 1, D), wsr)

    grid_spec1 = pltpu.PrefetchScalarGridSpec(
        num_scalar_prefetch=3,
        grid=(NB,),
        in_specs=[
            pl.BlockSpec((BLK, D), lambda b, be_r, ru, re: (b, 0)),
            pl.BlockSpec(memory_space=pl.ANY),
            pl.BlockSpec((1, 1, FT2),
                         lambda b, be_r, ru, re: (emin(be_r, b) * NF2 + 1,
                                                  0, 0)),
            pl.BlockSpec(memory_space=pl.ANY),
            pl.BlockSpec((BLK, D), lambda b, be_r, ru, re: (b, 0)),
            pl.BlockSpec((1, BLK, 128), lambda b, be_r, ru, re: (b, 0, 0)),
        ],
        out_specs=pl.BlockSpec((BLK, D), lambda b, be_r, ru, re: (b, 0)),
        scratch_shapes=ffn_scratch,
    )
    yp = pl.pallas_call(
        _ffn_kernel1,
        grid_spec=grid_spec1,
        out_shape=jax.ShapeDtypeStruct((NPAD, D), jnp.float32),
        compiler_params=pltpu.CompilerParams(
            dimension_semantics=("arbitrary",),
        ),
    )(blk_e, run_a, re_a, xs, W1, b1r, W2, yp0, wsr)

    out = pl.kernel(
        _combine_kernel,
        out_type=jax.ShapeDtypeStruct((T, D), jnp.float32),
        mesh=mesh,
        scratch_types=[
            pltpu.VMEM((TPT,), jnp.int32),
            pltpu.VMEM((TPT,), jnp.int32),
            pltpu.VMEM((CCH, D), jnp.float32),
            pltpu.VMEM((CCH, D), jnp.float32),
            pltpu.VMEM((CCH, D), jnp.float32),
            pltpu.VMEM((CCH, D), jnp.float32),
            pltpu.VMEM((CCH, D), jnp.float32),
            pltpu.VMEM((CCH, D), jnp.float32),
            pltpu.VMEM((CCH, D), jnp.float32),
            pltpu.VMEM((CCH, D), jnp.float32),
            [pltpu.SemaphoreType.DMA] * 8,
        ],
    )(flat, yp, pos0, pos1)

    return out.reshape(b, s, d)
